# Initial kernel scaffold; baseline (speedup 1.0000x reference)
import jax
import jax.numpy as jnp
from jax.experimental import pallas as pl


def kernel(x, edge_index, W1, b1, W2, b2, Wf1, bf1, Wf2, bf2, Wf3, bf3, Ws, bs):
    raise NotImplementedError("write your pallas kernel here")


# speedup vs baseline: 6.7565x; 6.7565x over previous
import functools

import jax
import jax.numpy as jnp
from jax import lax
from jax.experimental import pallas as pl
from jax.experimental.pallas import tpu as pltpu
from jax.experimental.pallas import tpu_sc as plsc

N = 10000
E = 320000
D = 128
P = 10240
ROWS = P // 128
NC = 2
NS = 16
NW = NC * NS
EPW = P
CHUNKS = EPW // 128
TPR = P // NS

_mesh = plsc.VectorSubcoreMesh(core_axis_name="c", subcore_axis_name="s")
_sc_params = pltpu.CompilerParams(needs_layout_passes=False)


@functools.partial(
    pl.kernel,
    out_type=jax.ShapeDtypeStruct((NW, P), jnp.float32),
    mesh=_mesh,
    scratch_types=[
        pltpu.VMEM((CHUNKS, 128), jnp.int32),
        pltpu.VMEM((P,), jnp.float32),
    ],
    compiler_params=_sc_params,
)
def _count_kernel(dst_hbm, out_hbm, dst_v, cnt_v):
  c = lax.axis_index("c")
  s = lax.axis_index("s")
  wid = s * NC + c
  pltpu.sync_copy(dst_hbm.at[wid], dst_v)

  zeros16 = jnp.zeros((16,), jnp.float32)

  def _zero(i, carry):
    cnt_v[pl.ds(i * 16, 16)] = zeros16
    return carry

  lax.fori_loop(0, P // 16, _zero, 0)

  ones16 = jnp.ones((16,), jnp.float32)

  def _count(i, carry):
    for t in range(8):
      idx = dst_v[i, pl.ds(t * 16, 16)]
      plsc.addupdate_scatter(cnt_v, [idx], ones16)
    return carry

  lax.fori_loop(0, CHUNKS, _count, 0)
  pltpu.sync_copy(cnt_v, out_hbm.at[wid])


@functools.partial(
    pl.kernel,
    out_type=jax.ShapeDtypeStruct((NC, P, 128), jnp.float32),
    mesh=_mesh,
    scratch_types=[
        pltpu.VMEM((CHUNKS, 128), jnp.int32),
        pltpu.VMEM((CHUNKS, 128), jnp.int32),
        pltpu.VMEM((128, 128), jnp.float32),
        pltpu.VMEM_SHARED((P, 128), jnp.float32),
        pltpu.SemaphoreType.DMA,
    ],
    compiler_params=_sc_params,
)
def _edge_kernel(g_hbm, src_hbm, dst_hbm, z_hbm, out_hbm,
                 src_v, dst_v, rows_v, acc, sem):
  c = lax.axis_index("c")
  s = lax.axis_index("s")
  wid = s * NC + c
  pltpu.sync_copy(src_hbm.at[wid], src_v)
  pltpu.sync_copy(dst_hbm.at[wid], dst_v)
  pltpu.sync_copy(z_hbm, acc.at[pl.ds(s * TPR, TPR)])
  plsc.subcore_barrier()

  def _body(j, carry):
    pltpu.async_copy(g_hbm.at[src_v.at[j]], rows_v, sem).wait()
    pltpu.sync_copy(rows_v, acc.at[dst_v.at[j]], add=True)
    return carry

  lax.fori_loop(0, CHUNKS, _body, 0)
  plsc.subcore_barrier()
  pltpu.sync_copy(acc.at[pl.ds(s * TPR, TPR)],
                  out_hbm.at[c, pl.ds(s * TPR, TPR)])


def _inv_from_counts(cp):
  return lax.rsqrt(jnp.sum(cp, axis=0) + 1.0)


def _layer1_body(x_ref, w_ref, b_ref, cp_ref, o_ref):
  inv = _inv_from_counts(cp_ref[...])
  hw = jnp.dot(x_ref[...], w_ref[...], preferred_element_type=jnp.float32)
  o_ref[...] = (hw + b_ref[...]) * inv


def _layer2_body(p0_ref, p1_ref, g_ref, cp_ref, w_ref, b_ref, o_ref):
  inv = _inv_from_counts(cp_ref[...])
  h = jax.nn.relu(inv * (p0_ref[...] + p1_ref[...] + g_ref[...]))
  hw = jnp.dot(h, w_ref[...], preferred_element_type=jnp.float32)
  o_ref[...] = (hw + b_ref[...]) * inv


def _ff_body(p0_ref, p1_ref, g_ref, cp_ref, wf1_ref, bf1_ref, wf2_ref, bf2_ref,
             wf3_ref, bf3_ref, ws_ref, bs_ref, o_ref):
  inv = _inv_from_counts(cp_ref[...])
  hx = inv * (p0_ref[...] + p1_ref[...] + g_ref[...])
  h = jax.nn.relu(jnp.dot(hx, wf1_ref[...],
                          preferred_element_type=jnp.float32) + bf1_ref[...])
  h = jax.nn.relu(jnp.dot(h, wf2_ref[...],
                          preferred_element_type=jnp.float32) + bf2_ref[...])
  h = jax.nn.relu(jnp.dot(h, wf3_ref[...],
                          preferred_element_type=jnp.float32) + bf3_ref[...])
  o_ref[...] = h + jnp.dot(hx, ws_ref[...],
                           preferred_element_type=jnp.float32) + bs_ref[...]


_BLK = 512
_GRID = P // _BLK

_node_spec = pl.BlockSpec((_BLK, 128), lambda i: (i, 0))
_w_spec = pl.BlockSpec((128, 128), lambda i: (0, 0))
_b_spec = pl.BlockSpec((1, 128), lambda i: (0, 0))
_cp_spec = pl.BlockSpec((NW, _BLK, 1), lambda i: (0, i, 0))


def _tc_call(body, n_in):
  return pl.pallas_call(
      body,
      grid=(_GRID,),
      in_specs=n_in,
      out_specs=_node_spec,
      out_shape=jax.ShapeDtypeStruct((P, 128), jnp.float32),
  )


def kernel(x, edge_index, W1, b1, W2, b2, Wf1, bf1, Wf2, bf2, Wf3, bf3, Ws, bs):
  src = edge_index[0]
  dst = edge_index[1]
  pad = NW * EPW - E
  src_p = jnp.concatenate([src, jnp.zeros((pad,), jnp.int32)]) \
      .reshape(NW, CHUNKS, 128)
  dst_p = jnp.concatenate([dst, jnp.full((pad,), N, jnp.int32)]) \
      .reshape(NW, CHUNKS, 128)
  x_p = jnp.concatenate([x, jnp.zeros((P - N, D), jnp.float32)], axis=0)
  zeros_tile = jnp.zeros((TPR, 128), jnp.float32)

  cp = _count_kernel(dst_p)
  cp = cp.reshape(NW, P, 1)

  b1r = b1.reshape(1, 128)
  b2r = b2.reshape(1, 128)

  g1 = _tc_call(_layer1_body,
                [_node_spec, _w_spec, _b_spec, _cp_spec])(x_p, W1, b1r, cp)
  p1 = _edge_kernel(g1, src_p, dst_p, zeros_tile)
  g2 = _tc_call(_layer2_body,
                [_node_spec, _node_spec, _node_spec, _cp_spec,
                 _w_spec, _b_spec])(p1[0], p1[1], g1, cp, W2, b2r)
  p2 = _edge_kernel(g2, src_p, dst_p, zeros_tile)
  out = _tc_call(_ff_body,
                 [_node_spec, _node_spec, _node_spec, _cp_spec,
                  _w_spec, _b_spec, _w_spec, _b_spec,
                  _w_spec, _b_spec, _w_spec, _b_spec])(
      p2[0], p2[1], g2, cp,
      Wf1, bf1.reshape(1, 128), Wf2, bf2.reshape(1, 128),
      Wf3, bf3.reshape(1, 128), Ws, bs.reshape(1, 128))
  return out[:N]


# speedup vs baseline: 7.6116x; 1.1266x over previous
import functools

import jax
import jax.numpy as jnp
from jax import lax
from jax.experimental import pallas as pl
from jax.experimental.pallas import tpu as pltpu
from jax.experimental.pallas import tpu_sc as plsc

N = 10000
E = 320000
D = 128
P = 10240
ROWS = P // 128
NC = 2
NS = 16
NW = NC * NS
EPW = P
CW = 128
CHUNKS = EPW // CW
NBLK = 2
BROWS = CHUNKS // NBLK
TPR = P // NS

_mesh = plsc.VectorSubcoreMesh(core_axis_name="c", subcore_axis_name="s")
_sc_params = pltpu.CompilerParams(needs_layout_passes=False)


@functools.partial(
    pl.kernel,
    out_type=jax.ShapeDtypeStruct((NW, P), jnp.float32),
    mesh=_mesh,
    scratch_types=[
        pltpu.VMEM((CHUNKS, CW), jnp.int32),
        pltpu.VMEM((P,), jnp.float32),
    ],
    compiler_params=_sc_params,
)
def _count_kernel(dst_hbm, out_hbm, dst_v, cnt_v):
  c = lax.axis_index("c")
  s = lax.axis_index("s")
  wid = s * NC + c
  pltpu.sync_copy(dst_hbm.at[wid], dst_v)

  zeros16 = jnp.zeros((16,), jnp.float32)

  def _zero(i, carry):
    cnt_v[pl.ds(i * 16, 16)] = zeros16
    return carry

  lax.fori_loop(0, P // 16, _zero, 0)

  ones16 = jnp.ones((16,), jnp.float32)

  def _count(i, carry):
    for t in range(CW // 16):
      idx = dst_v[i, pl.ds(t * 16, 16)]
      plsc.addupdate_scatter(cnt_v, [idx], ones16)
    return carry

  lax.fori_loop(0, CHUNKS, _count, 0)
  pltpu.sync_copy(cnt_v, out_hbm.at[wid])


@functools.partial(
    pl.kernel,
    out_type=jax.ShapeDtypeStruct((NC, P, 128), jnp.float32),
    mesh=_mesh,
    scratch_types=[
        pltpu.VMEM((BROWS, CW), jnp.int32),
        pltpu.VMEM((BROWS, CW), jnp.int32),
        pltpu.VMEM((CW, 128), jnp.float32),
        pltpu.VMEM((CW, 128), jnp.float32),
        pltpu.VMEM_SHARED((P, 128), jnp.float32),
        pltpu.SemaphoreType.DMA,
        pltpu.SemaphoreType.DMA,
    ],
    compiler_params=_sc_params,
)
def _edge_kernel(g_hbm, src_hbm, dst_hbm, z_hbm, out_hbm,
                 src_v, dst_v, rows0_v, rows1_v, acc, sem0, sem1):
  c = lax.axis_index("c")
  s = lax.axis_index("s")
  wid = s * NC + c
  pltpu.sync_copy(z_hbm, acc.at[pl.ds(s * TPR, TPR)])
  plsc.subcore_barrier()

  for blk in range(NBLK):
    pltpu.sync_copy(src_hbm.at[wid, pl.ds(blk * BROWS, BROWS)], src_v)
    pltpu.sync_copy(dst_hbm.at[wid, pl.ds(blk * BROWS, BROWS)], dst_v)
    pltpu.async_copy(g_hbm.at[src_v.at[0]], rows0_v, sem0)

    def _body(i, carry):
      j0 = 2 * i
      pltpu.async_copy(g_hbm.at[src_v.at[j0 + 1]], rows1_v, sem1)
      pltpu.make_async_copy(g_hbm.at[src_v.at[j0]], rows0_v, sem0).wait()
      pltpu.sync_copy(rows0_v, acc.at[dst_v.at[j0]], add=True)

      @pl.when(i < BROWS // 2 - 1)
      def _():
        pltpu.async_copy(g_hbm.at[src_v.at[j0 + 2]], rows0_v, sem0)

      pltpu.make_async_copy(g_hbm.at[src_v.at[j0 + 1]], rows1_v, sem1).wait()
      pltpu.sync_copy(rows1_v, acc.at[dst_v.at[j0 + 1]], add=True)
      return carry

    lax.fori_loop(0, BROWS // 2, _body, 0)
  plsc.subcore_barrier()
  pltpu.sync_copy(acc.at[pl.ds(s * TPR, TPR)],
                  out_hbm.at[c, pl.ds(s * TPR, TPR)])


def _inv_from_counts(cp):
  return lax.rsqrt(jnp.sum(cp, axis=0) + 1.0)


def _layer1_body(x_ref, w_ref, b_ref, cp_ref, o_ref):
  inv = _inv_from_counts(cp_ref[...])
  hw = jnp.dot(x_ref[...], w_ref[...], preferred_element_type=jnp.float32)
  o_ref[...] = (hw + b_ref[...]) * inv


def _layer2_body(p0_ref, p1_ref, g_ref, cp_ref, w_ref, b_ref, o_ref):
  inv = _inv_from_counts(cp_ref[...])
  h = jax.nn.relu(inv * (p0_ref[...] + p1_ref[...] + g_ref[...]))
  hw = jnp.dot(h, w_ref[...], preferred_element_type=jnp.float32)
  o_ref[...] = (hw + b_ref[...]) * inv


def _ff_body(p0_ref, p1_ref, g_ref, cp_ref, wf1_ref, bf1_ref, wf2_ref, bf2_ref,
             wf3_ref, bf3_ref, ws_ref, bs_ref, o_ref):
  inv = _inv_from_counts(cp_ref[...])
  hx = inv * (p0_ref[...] + p1_ref[...] + g_ref[...])
  h = jax.nn.relu(jnp.dot(hx, wf1_ref[...],
                          preferred_element_type=jnp.float32) + bf1_ref[...])
  h = jax.nn.relu(jnp.dot(h, wf2_ref[...],
                          preferred_element_type=jnp.float32) + bf2_ref[...])
  h = jax.nn.relu(jnp.dot(h, wf3_ref[...],
                          preferred_element_type=jnp.float32) + bf3_ref[...])
  o_ref[...] = h + jnp.dot(hx, ws_ref[...],
                           preferred_element_type=jnp.float32) + bs_ref[...]


_BLK = 512
_GRID = P // _BLK

_node_spec = pl.BlockSpec((_BLK, 128), lambda i: (i, 0))
_w_spec = pl.BlockSpec((128, 128), lambda i: (0, 0))
_b_spec = pl.BlockSpec((1, 128), lambda i: (0, 0))
_cp_spec = pl.BlockSpec((NW, _BLK, 1), lambda i: (0, i, 0))


def _tc_call(body, n_in):
  return pl.pallas_call(
      body,
      grid=(_GRID,),
      in_specs=n_in,
      out_specs=_node_spec,
      out_shape=jax.ShapeDtypeStruct((P, 128), jnp.float32),
  )


def kernel(x, edge_index, W1, b1, W2, b2, Wf1, bf1, Wf2, bf2, Wf3, bf3, Ws, bs):
  src = edge_index[0]
  dst = edge_index[1]
  pad = NW * EPW - E
  src_p = jnp.concatenate([src, jnp.zeros((pad,), jnp.int32)]) \
      .reshape(NW, CHUNKS, CW)
  dst_p = jnp.concatenate([dst, jnp.full((pad,), N, jnp.int32)]) \
      .reshape(NW, CHUNKS, CW)
  x_p = jnp.concatenate([x, jnp.zeros((P - N, D), jnp.float32)], axis=0)
  zeros_tile = jnp.zeros((TPR, 128), jnp.float32)

  cp = _count_kernel(dst_p)
  cp = cp.reshape(NW, P, 1)

  b1r = b1.reshape(1, 128)
  b2r = b2.reshape(1, 128)

  g1 = _tc_call(_layer1_body,
                [_node_spec, _w_spec, _b_spec, _cp_spec])(x_p, W1, b1r, cp)
  p1 = _edge_kernel(g1, src_p, dst_p, zeros_tile)
  g2 = _tc_call(_layer2_body,
                [_node_spec, _node_spec, _node_spec, _cp_spec,
                 _w_spec, _b_spec])(p1[0], p1[1], g1, cp, W2, b2r)
  p2 = _edge_kernel(g2, src_p, dst_p, zeros_tile)
  out = _tc_call(_ff_body,
                 [_node_spec, _node_spec, _node_spec, _cp_spec,
                  _w_spec, _b_spec, _w_spec, _b_spec,
                  _w_spec, _b_spec, _w_spec, _b_spec])(
      p2[0], p2[1], g2, cp,
      Wf1, bf1.reshape(1, 128), Wf2, bf2.reshape(1, 128),
      Wf3, bf3.reshape(1, 128), Ws, bs.reshape(1, 128))
  return out[:N]


# speedup vs baseline: 7.6455x; 1.0045x over previous
import functools

import jax
import jax.numpy as jnp
from jax import lax
from jax.experimental import pallas as pl
from jax.experimental.pallas import tpu as pltpu
from jax.experimental.pallas import tpu_sc as plsc

N = 10000
E = 320000
D = 128
P = 10240
ROWS = P // 128
NC = 2
NS = 16
NW = NC * NS
EPW = P
CW = 128
CHUNKS = EPW // CW
NBLK = 2
BROWS = CHUNKS // NBLK
TPR = P // NS

_mesh = plsc.VectorSubcoreMesh(core_axis_name="c", subcore_axis_name="s")
_sc_params = pltpu.CompilerParams(needs_layout_passes=False)


@functools.partial(
    pl.kernel,
    out_type=jax.ShapeDtypeStruct((NW, P), jnp.float32),
    mesh=_mesh,
    scratch_types=[
        pltpu.VMEM((CHUNKS, CW), jnp.int32),
        pltpu.VMEM((P,), jnp.float32),
    ],
    compiler_params=_sc_params,
)
def _count_kernel(dst_hbm, out_hbm, dst_v, cnt_v):
  c = lax.axis_index("c")
  s = lax.axis_index("s")
  wid = s * NC + c
  pltpu.sync_copy(dst_hbm.at[wid], dst_v)

  zeros16 = jnp.zeros((16,), jnp.float32)

  def _zero(i, carry):
    cnt_v[pl.ds(i * 16, 16)] = zeros16
    return carry

  lax.fori_loop(0, P // 16, _zero, 0)

  ones16 = jnp.ones((16,), jnp.float32)

  def _count(i, carry):
    for t in range(CW // 16):
      idx = dst_v[i, pl.ds(t * 16, 16)]
      plsc.addupdate_scatter(cnt_v, [idx], ones16)
    return carry

  lax.fori_loop(0, CHUNKS, _count, 0)
  pltpu.sync_copy(cnt_v, out_hbm.at[wid])


@functools.partial(
    pl.kernel,
    out_type=jax.ShapeDtypeStruct((NC, P, 128), jnp.float32),
    mesh=_mesh,
    scratch_types=[
        pltpu.VMEM((BROWS, CW), jnp.int32),
        pltpu.VMEM((BROWS, CW), jnp.int32),
        pltpu.VMEM((CW, 128), jnp.float32),
        pltpu.VMEM((CW, 128), jnp.float32),
        pltpu.VMEM_SHARED((P, 128), jnp.float32),
        pltpu.SemaphoreType.DMA,
        pltpu.SemaphoreType.DMA,
    ],
    compiler_params=_sc_params,
)
def _edge_kernel(g_hbm, src_hbm, dst_hbm, z_hbm, out_hbm,
                 src_v, dst_v, rows0_v, rows1_v, acc, sem0, sem1):
  c = lax.axis_index("c")
  s = lax.axis_index("s")
  wid = s * NC + c
  pltpu.sync_copy(z_hbm, acc.at[pl.ds(s * TPR, TPR)])
  plsc.subcore_barrier()

  for blk in range(NBLK):
    pltpu.sync_copy(src_hbm.at[wid, pl.ds(blk * BROWS, BROWS)], src_v)
    pltpu.sync_copy(dst_hbm.at[wid, pl.ds(blk * BROWS, BROWS)], dst_v)
    pltpu.async_copy(g_hbm.at[src_v.at[0]], rows0_v, sem0)

    def _body(i, carry):
      j0 = 2 * i
      pltpu.async_copy(g_hbm.at[src_v.at[j0 + 1]], rows1_v, sem1)
      pltpu.make_async_copy(g_hbm.at[src_v.at[j0]], rows0_v, sem0).wait()
      pltpu.sync_copy(rows0_v, acc.at[dst_v.at[j0]], add=True)

      @pl.when(i < BROWS // 2 - 1)
      def _():
        pltpu.async_copy(g_hbm.at[src_v.at[j0 + 2]], rows0_v, sem0)

      pltpu.make_async_copy(g_hbm.at[src_v.at[j0 + 1]], rows1_v, sem1).wait()
      pltpu.sync_copy(rows1_v, acc.at[dst_v.at[j0 + 1]], add=True)
      return carry

    lax.fori_loop(0, BROWS // 2, _body, 0)
  plsc.subcore_barrier()
  pltpu.sync_copy(acc.at[pl.ds(s * TPR, TPR)],
                  out_hbm.at[c, pl.ds(s * TPR, TPR)])


_ONES32 = None


def _inv_from_counts(cp):
  ones = jnp.ones((NW, 1), jnp.float32)
  tot = lax.dot_general(cp, ones, (((0,), (0,)), ((), ())),
                        preferred_element_type=jnp.float32)
  return lax.rsqrt(tot + 1.0)


def _layer1_body(x_ref, w_ref, b_ref, cp_ref, o_ref):
  inv = _inv_from_counts(cp_ref[...])
  hw = jnp.dot(x_ref[...], w_ref[...], preferred_element_type=jnp.float32)
  o_ref[...] = (hw + b_ref[...]) * inv


def _layer2_body(p0_ref, p1_ref, g_ref, cp_ref, w_ref, b_ref, o_ref):
  inv = _inv_from_counts(cp_ref[...])
  h = jax.nn.relu(inv * (p0_ref[...] + p1_ref[...] + g_ref[...]))
  hw = jnp.dot(h, w_ref[...], preferred_element_type=jnp.float32)
  o_ref[...] = (hw + b_ref[...]) * inv


def _ff_body(p0_ref, p1_ref, g_ref, cp_ref, wf1_ref, bf1_ref, wf2_ref, bf2_ref,
             wf3_ref, bf3_ref, ws_ref, bs_ref, o_ref):
  inv = _inv_from_counts(cp_ref[...])
  hx = inv * (p0_ref[...] + p1_ref[...] + g_ref[...])
  h = jax.nn.relu(jnp.dot(hx, wf1_ref[...],
                          preferred_element_type=jnp.float32) + bf1_ref[...])
  h = jax.nn.relu(jnp.dot(h, wf2_ref[...],
                          preferred_element_type=jnp.float32) + bf2_ref[...])
  h = jax.nn.relu(jnp.dot(h, wf3_ref[...],
                          preferred_element_type=jnp.float32) + bf3_ref[...])
  o_ref[...] = h + jnp.dot(hx, ws_ref[...],
                           preferred_element_type=jnp.float32) + bs_ref[...]


_BLK = 512
_GRID = P // _BLK

_node_spec = pl.BlockSpec((_BLK, 128), lambda i: (i, 0))
_w_spec = pl.BlockSpec((128, 128), lambda i: (0, 0))
_b_spec = pl.BlockSpec((1, 128), lambda i: (0, 0))
_cp_spec = pl.BlockSpec((NW, _BLK), lambda i: (0, i))


def _tc_call(body, n_in):
  return pl.pallas_call(
      body,
      grid=(_GRID,),
      in_specs=n_in,
      out_specs=_node_spec,
      out_shape=jax.ShapeDtypeStruct((P, 128), jnp.float32),
  )


def kernel(x, edge_index, W1, b1, W2, b2, Wf1, bf1, Wf2, bf2, Wf3, bf3, Ws, bs):
  src = edge_index[0]
  dst = edge_index[1]
  pad = NW * EPW - E
  src_p = jnp.concatenate([src, jnp.zeros((pad,), jnp.int32)]) \
      .reshape(NW, CHUNKS, CW)
  dst_p = jnp.concatenate([dst, jnp.full((pad,), N, jnp.int32)]) \
      .reshape(NW, CHUNKS, CW)
  x_p = jnp.concatenate([x, jnp.zeros((P - N, D), jnp.float32)], axis=0)
  zeros_tile = jnp.zeros((TPR, 128), jnp.float32)

  cp = _count_kernel(dst_p)

  b1r = b1.reshape(1, 128)
  b2r = b2.reshape(1, 128)

  g1 = _tc_call(_layer1_body,
                [_node_spec, _w_spec, _b_spec, _cp_spec])(x_p, W1, b1r, cp)
  p1 = _edge_kernel(g1, src_p, dst_p, zeros_tile)
  g2 = _tc_call(_layer2_body,
                [_node_spec, _node_spec, _node_spec, _cp_spec,
                 _w_spec, _b_spec])(p1[0], p1[1], g1, cp, W2, b2r)
  p2 = _edge_kernel(g2, src_p, dst_p, zeros_tile)
  out = _tc_call(_ff_body,
                 [_node_spec, _node_spec, _node_spec, _cp_spec,
                  _w_spec, _b_spec, _w_spec, _b_spec,
                  _w_spec, _b_spec, _w_spec, _b_spec])(
      p2[0], p2[1], g2, cp,
      Wf1, bf1.reshape(1, 128), Wf2, bf2.reshape(1, 128),
      Wf3, bf3.reshape(1, 128), Ws, bs.reshape(1, 128))
  return out[:N]


# speedup vs baseline: 9.6902x; 1.2674x over previous
import functools

import jax
import jax.numpy as jnp
from jax import lax
from jax.experimental import pallas as pl
from jax.experimental.pallas import tpu as pltpu
from jax.experimental.pallas import tpu_sc as plsc

N = 10000
E = 320000
D = 128
P = 10240
ROWS = P // 128
NC = 2
NS = 16
NW = NC * NS
CW = 128
TOTCH = 2560
BROWS = 40
CH_FAST = 120
CH_SLOW = 40
NB_FAST = CH_FAST // BROWS
NB_SLOW = CH_SLOW // BROWS
FAST_CORE = 0
TPR = P // NS

_mesh = plsc.VectorSubcoreMesh(core_axis_name="c", subcore_axis_name="s")
_sc_params = pltpu.CompilerParams(needs_layout_passes=False)


@functools.partial(
    pl.kernel,
    out_type=jax.ShapeDtypeStruct((NW, P), jnp.float32),
    mesh=_mesh,
    scratch_types=[
        pltpu.VMEM((CH_FAST, CW), jnp.int32),
        pltpu.VMEM((P,), jnp.float32),
    ],
    compiler_params=_sc_params,
)
def _count_kernel(dst_hbm, out_hbm, dst_v, cnt_v):
  c = lax.axis_index("c")
  s = lax.axis_index("s")
  wid = s * NC + c
  nch = jnp.where(c == FAST_CORE, CH_FAST, CH_SLOW)
  pltpu.sync_copy(dst_hbm.at[wid], dst_v)

  zeros16 = jnp.zeros((16,), jnp.float32)

  def _zero(i, carry):
    cnt_v[pl.ds(i * 16, 16)] = zeros16
    return carry

  lax.fori_loop(0, P // 16, _zero, 0)

  ones16 = jnp.ones((16,), jnp.float32)

  def _count(i, carry):
    for t in range(CW // 16):
      idx = dst_v[i, pl.ds(t * 16, 16)]
      plsc.addupdate_scatter(cnt_v, [idx], ones16)
    return carry

  lax.fori_loop(0, nch, _count, 0)
  pltpu.sync_copy(cnt_v, out_hbm.at[wid])


@functools.partial(
    pl.kernel,
    out_type=jax.ShapeDtypeStruct((NC, P, 128), jnp.float32),
    mesh=_mesh,
    scratch_types=[
        pltpu.VMEM((BROWS, CW), jnp.int32),
        pltpu.VMEM((BROWS, CW), jnp.int32),
        pltpu.VMEM((CW, 128), jnp.float32),
        pltpu.VMEM((CW, 128), jnp.float32),
        pltpu.VMEM_SHARED((P, 128), jnp.float32),
        pltpu.SemaphoreType.DMA,
        pltpu.SemaphoreType.DMA,
    ],
    compiler_params=_sc_params,
)
def _edge_kernel(g_hbm, src_hbm, dst_hbm, z_hbm, out_hbm,
                 src_v, dst_v, rows0_v, rows1_v, acc, sem0, sem1):
  c = lax.axis_index("c")
  s = lax.axis_index("s")
  wid = s * NC + c
  pltpu.sync_copy(z_hbm, acc.at[pl.ds(s * TPR, TPR)])
  plsc.subcore_barrier()

  nblk = jnp.where(c == FAST_CORE, NB_FAST, NB_SLOW)

  def _block(blk, bcarry):
    pltpu.sync_copy(src_hbm.at[wid, pl.ds(blk * BROWS, BROWS)], src_v)
    pltpu.sync_copy(dst_hbm.at[wid, pl.ds(blk * BROWS, BROWS)], dst_v)
    pltpu.async_copy(g_hbm.at[src_v.at[0]], rows0_v, sem0)

    def _body(i, carry):
      j0 = 2 * i
      pltpu.async_copy(g_hbm.at[src_v.at[j0 + 1]], rows1_v, sem1)
      pltpu.make_async_copy(g_hbm.at[src_v.at[j0]], rows0_v, sem0).wait()
      pltpu.sync_copy(rows0_v, acc.at[dst_v.at[j0]], add=True)

      @pl.when(i < BROWS // 2 - 1)
      def _():
        pltpu.async_copy(g_hbm.at[src_v.at[j0 + 2]], rows0_v, sem0)

      pltpu.make_async_copy(g_hbm.at[src_v.at[j0 + 1]], rows1_v, sem1).wait()
      pltpu.sync_copy(rows1_v, acc.at[dst_v.at[j0 + 1]], add=True)
      return carry

    lax.fori_loop(0, BROWS // 2, _body, 0)
    return bcarry

  lax.fori_loop(0, nblk, _block, 0)
  plsc.subcore_barrier()
  pltpu.sync_copy(acc.at[pl.ds(s * TPR, TPR)],
                  out_hbm.at[c, pl.ds(s * TPR, TPR)])


_ONES32 = None


def _inv_from_counts(cp):
  ones = jnp.ones((NW, 1), jnp.float32)
  tot = lax.dot_general(cp, ones, (((0,), (0,)), ((), ())),
                        preferred_element_type=jnp.float32)
  return lax.rsqrt(tot + 1.0)


def _layer1_body(x_ref, w_ref, b_ref, cp_ref, o_ref):
  inv = _inv_from_counts(cp_ref[...])
  hw = jnp.dot(x_ref[...], w_ref[...], preferred_element_type=jnp.float32)
  o_ref[...] = (hw + b_ref[...]) * inv


def _layer2_body(p0_ref, p1_ref, g_ref, cp_ref, w_ref, b_ref, o_ref):
  inv = _inv_from_counts(cp_ref[...])
  h = jax.nn.relu(inv * (p0_ref[...] + p1_ref[...] + g_ref[...]))
  hw = jnp.dot(h, w_ref[...], preferred_element_type=jnp.float32)
  o_ref[...] = (hw + b_ref[...]) * inv


def _ff_body(p0_ref, p1_ref, g_ref, cp_ref, wf1_ref, bf1_ref, wf2_ref, bf2_ref,
             wf3_ref, bf3_ref, ws_ref, bs_ref, o_ref):
  inv = _inv_from_counts(cp_ref[...])
  hx = inv * (p0_ref[...] + p1_ref[...] + g_ref[...])
  h = jax.nn.relu(jnp.dot(hx, wf1_ref[...],
                          preferred_element_type=jnp.float32) + bf1_ref[...])
  h = jax.nn.relu(jnp.dot(h, wf2_ref[...],
                          preferred_element_type=jnp.float32) + bf2_ref[...])
  h = jax.nn.relu(jnp.dot(h, wf3_ref[...],
                          preferred_element_type=jnp.float32) + bf3_ref[...])
  o_ref[...] = h + jnp.dot(hx, ws_ref[...],
                           preferred_element_type=jnp.float32) + bs_ref[...]


_BLK = 512
_GRID = P // _BLK

_node_spec = pl.BlockSpec((_BLK, 128), lambda i: (i, 0))
_w_spec = pl.BlockSpec((128, 128), lambda i: (0, 0))
_b_spec = pl.BlockSpec((1, 128), lambda i: (0, 0))
_cp_spec = pl.BlockSpec((NW, _BLK), lambda i: (0, i))


def _tc_call(body, n_in):
  return pl.pallas_call(
      body,
      grid=(_GRID,),
      in_specs=n_in,
      out_specs=_node_spec,
      out_shape=jax.ShapeDtypeStruct((P, 128), jnp.float32),
  )


def kernel(x, edge_index, W1, b1, W2, b2, Wf1, bf1, Wf2, bf2, Wf3, bf3, Ws, bs):
  src = edge_index[0]
  dst = edge_index[1]
  pad = TOTCH * CW - E

  def _edge_layout(v, fill):
    flat = jnp.concatenate([v, jnp.full((pad,), fill, jnp.int32)]) \
        .reshape(NS, CH_FAST + CH_SLOW, CW)
    fast = flat[:, :CH_FAST]
    slow = jnp.concatenate(
        [flat[:, CH_FAST:],
         jnp.zeros((NS, CH_FAST - CH_SLOW, CW), jnp.int32)], axis=1)
    pair = (fast, slow) if FAST_CORE == 0 else (slow, fast)
    return jnp.stack(pair, axis=1).reshape(NW, CH_FAST, CW)

  src_p = _edge_layout(src, 0)
  dst_p = _edge_layout(dst, N)
  x_p = jnp.concatenate([x, jnp.zeros((P - N, D), jnp.float32)], axis=0)
  zeros_tile = jnp.zeros((TPR, 128), jnp.float32)

  cp = _count_kernel(dst_p)

  b1r = b1.reshape(1, 128)
  b2r = b2.reshape(1, 128)

  g1 = _tc_call(_layer1_body,
                [_node_spec, _w_spec, _b_spec, _cp_spec])(x_p, W1, b1r, cp)
  p1 = _edge_kernel(g1, src_p, dst_p, zeros_tile)
  g2 = _tc_call(_layer2_body,
                [_node_spec, _node_spec, _node_spec, _cp_spec,
                 _w_spec, _b_spec])(p1[0], p1[1], g1, cp, W2, b2r)
  p2 = _edge_kernel(g2, src_p, dst_p, zeros_tile)
  out = _tc_call(_ff_body,
                 [_node_spec, _node_spec, _node_spec, _cp_spec,
                  _w_spec, _b_spec, _w_spec, _b_spec,
                  _w_spec, _b_spec, _w_spec, _b_spec])(
      p2[0], p2[1], g2, cp,
      Wf1, bf1.reshape(1, 128), Wf2, bf2.reshape(1, 128),
      Wf3, bf3.reshape(1, 128), Ws, bs.reshape(1, 128))
  return out[:N]
